# baseline (speedup 1.0000x reference)
import dataclasses
import functools

import jax
import jax.numpy as jnp
from jax import lax
from jax.experimental import pallas as pl
from jax.experimental.pallas import tpu as pltpu
from jax.experimental.pallas import tpu_sc as plsc

_NUM_CORES = 2
_NUM_SUBCORES = 16
_NUM_WORKERS = _NUM_CORES * _NUM_SUBCORES
_NBUF = 8
_LANES = 16


def kernel(table, indices):
    tt = table.T
    idx = jnp.squeeze(indices, axis=-1).astype(jnp.int32)
    num_idx = idx.shape[0]
    dim = table.shape[1]
    per_worker = num_idx // _NUM_WORKERS

    mesh = plsc.VectorSubcoreMesh(core_axis_name="c", subcore_axis_name="s")

    cp = pltpu.CompilerParams()
    if "needs_layout_passes" in pltpu.CompilerParams.__dataclass_fields__:
        cp = dataclasses.replace(cp, needs_layout_passes=False)

    @functools.partial(
        pl.kernel,
        mesh=mesh,
        compiler_params=cp,
        out_type=jax.ShapeDtypeStruct((dim, num_idx), jnp.float32),
        scratch_types=[
            pltpu.VMEM_SHARED((num_idx,), jnp.int32),
            pltpu.SMEM((per_worker,), jnp.int32),
            pltpu.VMEM((dim, num_idx // _NUM_WORKERS), jnp.float32),
        ]
        + [pltpu.VMEM((dim, 128), jnp.float32) for _ in range(_NBUF)]
        + [pltpu.SemaphoreType.DMA for _ in range(_NBUF)],
    )
    def gather_kernel(tt_hbm, idx_hbm, out_hbm, idx_sp, idx_s, gbuf, *rest):
        chunks = rest[:_NBUF]
        sems = rest[_NBUF : 2 * _NBUF]
        wid = lax.axis_index("s") * _NUM_CORES + lax.axis_index("c")
        base = wid * per_worker
        pltpu.sync_copy(
            idx_hbm.at[pl.ds(base, per_worker)], idx_sp.at[pl.ds(base, per_worker)]
        )
        pltpu.sync_copy(idx_sp.at[pl.ds(base, per_worker)], idx_s)

        def fire(j, slot, sem):
            r = idx_s[j]
            c = pl.multiple_of((r // 128) * 128, 128)
            for g in range(dim // 8):
                pltpu.async_copy(
                    tt_hbm.at[pl.ds(8 * g, 8), pl.ds(c, 128)],
                    slot.at[pl.ds(8 * g, 8), :],
                    sem,
                )

        for s in range(_NBUF):
            fire(s, chunks[s], sems[s])

        @pl.loop(0, per_worker, step=_NBUF)
        def _(j0):
            for s in range(_NBUF):
                j = j0 + s
                pltpu.make_async_copy(
                    tt_hbm.at[:, pl.ds(0, 128)], chunks[s], sems[s]
                ).wait()
                r = idx_s[j]
                lane = r - (r // 128) * 128
                lv = jnp.full((_LANES,), 0, jnp.int32) + lane
                jv = jnp.full((_LANES,), 0, jnp.int32) + j
                for g in range(dim // _LANES):
                    fv = lax.broadcasted_iota(jnp.int32, (_LANES,), 0) + g * _LANES
                    vals = plsc.load_gather(chunks[s], [fv, lv])
                    plsc.store_scatter(gbuf, [fv, jv], vals)

                @pl.when(j + _NBUF < per_worker)
                def _():
                    fire(j + _NBUF, chunks[s], sems[s])

        pltpu.sync_copy(gbuf, out_hbm.at[:, pl.ds(base, per_worker)])

    return gather_kernel(tt, idx).T
